# Initial kernel scaffold; baseline (speedup 1.0000x reference)
import jax
import jax.numpy as jnp
from jax.experimental import pallas as pl


def kernel(x, edge_index, perm, W1, b1, W2, b2, Wd):
    raise NotImplementedError("write your pallas kernel here")


# speedup vs baseline: 2.4640x; 2.4640x over previous
import functools

import jax
import jax.numpy as jnp
from jax import lax
from jax.experimental import pallas as pl
from jax.experimental.pallas import tpu as pltpu
from jax.experimental.pallas import tpu_sc as plsc

EPS = 1e-15

NC = 2
NT = 16
CHUNK = 128


def _mesh():
    return plsc.VectorSubcoreMesh(core_axis_name="c", subcore_axis_name="s")


def _sc_gather(table, idx_flat, npad, d):
    rows_per_w = npad // (NC * NT)
    nch = rows_per_w // 64

    @functools.partial(
        pl.kernel, mesh=_mesh(),
        out_type=jax.ShapeDtypeStruct((npad, d), jnp.float32),
        scratch_types=[
            pltpu.VMEM((64,), jnp.int32),
            pltpu.VMEM((64, d), jnp.float32),
            pltpu.SemaphoreType.DMA,
        ],
    )
    def k(tab_hbm, idx_hbm, out_hbm, idx_v, rows_v, sem):
        c = lax.axis_index("c")
        s = lax.axis_index("s")
        wid = c * NT + s

        def body(j, carry):
            pltpu.sync_copy(idx_hbm.at[pl.ds(wid * rows_per_w + j * 64, 64)], idx_v)
            pltpu.async_copy(tab_hbm.at[idx_v], rows_v, sem).wait()
            pltpu.sync_copy(rows_v, out_hbm.at[pl.ds(wid * rows_per_w + j * 64, 64)])
            return carry
        lax.fori_loop(0, nch, body, 0)

    return k(table, idx_flat)


def _tc_mm(x, w, npad, d, blk=1024):
    def body(x_ref, w_ref, o_ref):
        o_ref[:, :] = jnp.dot(x_ref[:, :], w_ref[:, :],
                              preferred_element_type=jnp.float32)
    return pl.pallas_call(
        body,
        grid=(npad // blk,),
        in_specs=[pl.BlockSpec((blk, d), lambda i: (i, 0)),
                  pl.BlockSpec((d, d), lambda i: (0, 0))],
        out_specs=pl.BlockSpec((blk, d), lambda i: (i, 0)),
        out_shape=jax.ShapeDtypeStruct((npad, d), jnp.float32),
    )(x, w)


def _tc_deg(dstb, npad, d, ne, K):
    nch = ne // K

    def body(dst_ref, out_ref, acc_ref):
        j = pl.program_id(0)

        @pl.when(j == 0)
        def _():
            acc_ref[:, :] = jnp.zeros_like(acc_ref)

        one_row = jnp.full((1, d), 1.0, jnp.float32)

        def eb(i, carry):
            di = dst_ref[i]
            acc_ref[pl.ds(di, 1), :] += one_row
            return carry
        lax.fori_loop(0, K, eb, 0, unroll=8)

        @pl.when(j == nch - 1)
        def _():
            out_ref[:, :] = lax.rsqrt(acc_ref[:, :] + 1.0)
    return pl.pallas_call(
        body,
        grid=(nch,),
        in_specs=[pl.BlockSpec((K,), lambda j: (j,),
                               memory_space=pltpu.SMEM)],
        out_specs=pl.BlockSpec((npad, d), lambda j: (0, 0)),
        out_shape=jax.ShapeDtypeStruct((npad, d), jnp.float32),
        scratch_shapes=[pltpu.VMEM((npad, d), jnp.float32)],
    )(dstb)


def _tc_scatter(tables, srcb, dstb, npad, d, ne, K):
    nch = ne // K

    def body(src_ref, dst_ref, tab_ref, out_ref, acc_ref):
        j = pl.program_id(1)

        @pl.when(j == 0)
        def _():
            acc_ref[:, :] = tab_ref[0]

        def eb(i, carry):
            si = src_ref[i]
            di = dst_ref[i]
            acc_ref[pl.ds(di, 1), :] += tab_ref[0, pl.ds(si, 1), :]
            return carry
        lax.fori_loop(0, K, eb, 0, unroll=8)

        @pl.when(j == nch - 1)
        def _():
            out_ref[0] = acc_ref[:, :]
    return pl.pallas_call(
        body,
        grid=(2, nch),
        in_specs=[pl.BlockSpec((K,), lambda c, j: (j,),
                               memory_space=pltpu.SMEM),
                  pl.BlockSpec((K,), lambda c, j: (j,),
                               memory_space=pltpu.SMEM),
                  pl.BlockSpec((1, npad, d), lambda c, j: (c, 0, 0))],
        out_specs=pl.BlockSpec((1, npad, d), lambda c, j: (c, 0, 0)),
        out_shape=jax.ShapeDtypeStruct((2, npad, d), jnp.float32),
        scratch_shapes=[pltpu.VMEM((npad, d), jnp.float32)],
    )(srcb, dstb, tables)


def _tc_scale(degp, h1, h1perm, npad, d, blk=512):
    def body(dinv_ref, h1_ref, hp_ref, t1_ref):
        dv = dinv_ref[:, :]
        t1_ref[0, :, :] = dv * h1_ref[:, :]
        t1_ref[1, :, :] = dv * hp_ref[:, :]
    return pl.pallas_call(
        body,
        grid=(npad // blk,),
        in_specs=[pl.BlockSpec((blk, d), lambda i: (i, 0)),
                  pl.BlockSpec((blk, d), lambda i: (i, 0)),
                  pl.BlockSpec((blk, d), lambda i: (i, 0))],
        out_specs=pl.BlockSpec((2, blk, d), lambda i: (0, i, 0)),
        out_shape=jax.ShapeDtypeStruct((2, npad, d), jnp.float32),
    )(degp, h1, h1perm)


def _tc_l1finish(acc1, dinv, b1, w2, n, npad, d, blk=512):
    def body(acc_ref, dinv_ref, b1_ref, w2_ref, t2_ref):
        i = pl.program_id(0)
        rows = lax.broadcasted_iota(jnp.int32, (blk, d), 0) + i * blk
        m = rows < n
        dv = dinv_ref[:, :]
        for cc in range(2):
            z = jnp.maximum(acc_ref[cc] * dv + b1_ref[:, :], 0.0)
            z = jnp.where(m, z, 0.0)
            t2_ref[cc, :, :] = jnp.dot(z, w2_ref[:, :],
                                       preferred_element_type=jnp.float32) * dv
    return pl.pallas_call(
        body,
        grid=(npad // blk,),
        in_specs=[pl.BlockSpec((2, blk, d), lambda i: (0, i, 0)),
                  pl.BlockSpec((blk, d), lambda i: (i, 0)),
                  pl.BlockSpec((1, d), lambda i: (0, 0)),
                  pl.BlockSpec((d, d), lambda i: (0, 0))],
        out_specs=pl.BlockSpec((2, blk, d), lambda i: (0, i, 0)),
        out_shape=jax.ShapeDtypeStruct((2, npad, d), jnp.float32),
    )(acc1, dinv, b1, w2)


def _tc_proj(acc2, dinv, b2, wd, n, npad, d, blk=512):
    def body(acc_ref, dinv_ref, b2_ref, wd_ref, proj_ref, sum_ref):
        i = pl.program_id(0)

        @pl.when(i == 0)
        def _():
            sum_ref[:, :] = jnp.zeros_like(sum_ref)

        rows = lax.broadcasted_iota(jnp.int32, (blk, d), 0) + i * blk
        z = acc_ref[0] * dinv_ref[:, :] + b2_ref[:, :]
        z = jnp.where(rows < n, z, 0.0)
        sum_ref[:, :] += jnp.sum(z, axis=0, keepdims=True)

        @pl.when(i == pl.num_programs(0) - 1)
        def _():
            summ = jax.nn.sigmoid(sum_ref[:, :] / float(n))
            proj_ref[:, :] = lax.dot_general(
                summ, wd_ref[:, :], (((1,), (1,)), ((), ())),
                preferred_element_type=jnp.float32)
    return pl.pallas_call(
        body,
        grid=(npad // blk,),
        in_specs=[pl.BlockSpec((1, blk, d), lambda i: (0, i, 0)),
                  pl.BlockSpec((blk, d), lambda i: (i, 0)),
                  pl.BlockSpec((1, d), lambda i: (0, 0)),
                  pl.BlockSpec((d, d), lambda i: (0, 0))],
        out_specs=pl.BlockSpec((1, d), lambda i: (0, 0)),
        out_shape=jax.ShapeDtypeStruct((1, d), jnp.float32),
        scratch_shapes=[pltpu.VMEM((1, d), jnp.float32)],
    )(acc2, dinv, b2, wd)


def _tc_loss(acc2, dinv, b2, proj, n, npad, d, blk=512):
    def body(acc_ref, dinv_ref, b2_ref, proj_ref, out_ref, s_ref):
        i = pl.program_id(0)

        @pl.when(i == 0)
        def _():
            s_ref[0] = 0.0
            s_ref[1] = 0.0

        rows = lax.broadcasted_iota(jnp.int32, (blk, 1), 0) + i * blk
        m = rows < n
        dv = dinv_ref[:, :]
        dn = (((1,), (1,)), ((), ()))
        pz = acc_ref[0] * dv + b2_ref[:, :]
        nz = acc_ref[1] * dv + b2_ref[:, :]
        sp = lax.dot_general(pz, proj_ref[:, :], dn,
                             preferred_element_type=jnp.float32)
        sn = lax.dot_general(nz, proj_ref[:, :], dn,
                             preferred_element_type=jnp.float32)
        lp = jnp.where(m, jnp.log(jax.nn.sigmoid(sp) + EPS), 0.0)
        ln = jnp.where(m, jnp.log(1.0 - jax.nn.sigmoid(sn) + EPS), 0.0)
        s_ref[0] += jnp.sum(lp)
        s_ref[1] += jnp.sum(ln)

        @pl.when(i == pl.num_programs(0) - 1)
        def _():
            out_ref[:, :] = jnp.full((1, 1), -(s_ref[0] + s_ref[1]) / float(n),
                                     jnp.float32)
    return pl.pallas_call(
        body,
        grid=(npad // blk,),
        in_specs=[pl.BlockSpec((2, blk, d), lambda i: (0, i, 0)),
                  pl.BlockSpec((blk, d), lambda i: (i, 0)),
                  pl.BlockSpec((1, d), lambda i: (0, 0)),
                  pl.BlockSpec((1, d), lambda i: (0, 0))],
        out_specs=pl.BlockSpec((1, 1), lambda i: (0, 0)),
        out_shape=jax.ShapeDtypeStruct((1, 1), jnp.float32),
        scratch_shapes=[pltpu.SMEM((2,), jnp.float32)],
    )(acc2, dinv, b2, proj)


def kernel(x, edge_index, perm, W1, b1, W2, b2, Wd):
    n, d = x.shape
    e = edge_index.shape[1]
    npad = ((n + 1 + 511) // 512) * 512

    src = edge_index[0].astype(jnp.int32)
    dst = edge_index[1].astype(jnp.int32)

    K = 16384
    ne = ((e + K - 1) // K) * K
    src_p = jnp.pad(src, (0, ne - e), constant_values=n)
    dst_p = jnp.pad(dst, (0, ne - e), constant_values=n)

    perm_p = jnp.pad(perm.astype(jnp.int32), (0, npad - n),
                     constant_values=n)
    x_p = jnp.pad(x, ((0, npad - n), (0, 0)))
    b1r = b1.reshape(1, d)
    b2r = b2.reshape(1, d)

    h1 = _tc_mm(x_p, W1, npad, d)
    dinv = _tc_deg(dst_p, npad, d, ne, K)
    h1perm = _sc_gather(h1, perm_p, npad, d)
    T1 = _tc_scale(dinv, h1, h1perm, npad, d)
    acc1 = _tc_scatter(T1, src_p, dst_p, npad, d, ne, K)
    T2 = _tc_l1finish(acc1, dinv, b1r, W2, n, npad, d)
    acc2 = _tc_scatter(T2, src_p, dst_p, npad, d, ne, K)
    proj = _tc_proj(acc2, dinv, b2r, Wd, n, npad, d)
    out = _tc_loss(acc2, dinv, b2r, proj, n, npad, d)
    return out.reshape(())


# speedup vs baseline: 3.4243x; 1.3897x over previous
import functools

import jax
import jax.numpy as jnp
from jax import lax
from jax.experimental import pallas as pl
from jax.experimental.pallas import tpu as pltpu
from jax.experimental.pallas import tpu_sc as plsc

EPS = 1e-15

NC = 2
NT = 16
CHUNK = 128


def _mesh():
    return plsc.VectorSubcoreMesh(core_axis_name="c", subcore_axis_name="s")


def _sc_gather(table, idx_flat, npad, d):
    rows_per_w = npad // (NC * NT)
    nch = rows_per_w // 64

    @functools.partial(
        pl.kernel, mesh=_mesh(),
        out_type=jax.ShapeDtypeStruct((npad, d), jnp.float32),
        scratch_types=[
            pltpu.VMEM((64,), jnp.int32),
            pltpu.VMEM((64, d), jnp.float32),
            pltpu.SemaphoreType.DMA,
        ],
    )
    def k(tab_hbm, idx_hbm, out_hbm, idx_v, rows_v, sem):
        c = lax.axis_index("c")
        s = lax.axis_index("s")
        wid = c * NT + s

        def body(j, carry):
            pltpu.sync_copy(idx_hbm.at[pl.ds(wid * rows_per_w + j * 64, 64)], idx_v)
            pltpu.async_copy(tab_hbm.at[idx_v], rows_v, sem).wait()
            pltpu.sync_copy(rows_v, out_hbm.at[pl.ds(wid * rows_per_w + j * 64, 64)])
            return carry
        lax.fori_loop(0, nch, body, 0)

    return k(table, idx_flat)


def _tc_mm(x, w, npad, d, blk=1024):
    def body(x_ref, w_ref, o_ref):
        o_ref[:, :] = jnp.dot(x_ref[:, :], w_ref[:, :],
                              preferred_element_type=jnp.float32)
    return pl.pallas_call(
        body,
        grid=(npad // blk,),
        in_specs=[pl.BlockSpec((blk, d), lambda i: (i, 0)),
                  pl.BlockSpec((d, d), lambda i: (0, 0))],
        out_specs=pl.BlockSpec((blk, d), lambda i: (i, 0)),
        out_shape=jax.ShapeDtypeStruct((npad, d), jnp.float32),
    )(x, w)


def _tc_deg(dstb, npad, d, ne, K):
    nch = ne // K

    def body(dst_ref, out_ref, acc_ref):
        j = pl.program_id(0)

        @pl.when(j == 0)
        def _():
            acc_ref[:, :] = jnp.zeros_like(acc_ref)

        one_row = jnp.full((1, d), 1.0, jnp.float32)

        def eb(i, carry):
            di = dst_ref[i]
            acc_ref[pl.ds(di, 1), :] += one_row
            return carry
        lax.fori_loop(0, K, eb, 0, unroll=8)

        @pl.when(j == nch - 1)
        def _():
            out_ref[:, :] = lax.rsqrt(acc_ref[:, :] + 1.0)
    return pl.pallas_call(
        body,
        grid=(nch,),
        in_specs=[pl.BlockSpec((K,), lambda j: (j,),
                               memory_space=pltpu.SMEM)],
        out_specs=pl.BlockSpec((npad, d), lambda j: (0, 0)),
        out_shape=jax.ShapeDtypeStruct((npad, d), jnp.float32),
        scratch_shapes=[pltpu.VMEM((npad, d), jnp.float32)],
    )(dstb)


def _tc_scatter(tables, srcb, dstb, npad, d, ne, K):
    nch = ne // K

    def body(src_ref, dst_ref, tab_ref, out_ref, acc0_ref, acc1_ref):
        j = pl.program_id(0)

        @pl.when(j == 0)
        def _():
            acc0_ref[:, :] = tab_ref[0]
            acc1_ref[:, :] = tab_ref[1]

        def eb(i, carry):
            si = src_ref[i]
            di = dst_ref[i]
            acc0_ref[pl.ds(di, 1), :] += tab_ref[0, pl.ds(si, 1), :]
            acc1_ref[pl.ds(di, 1), :] += tab_ref[1, pl.ds(si, 1), :]
            return carry
        lax.fori_loop(0, K, eb, 0, unroll=8)

        @pl.when(j == nch - 1)
        def _():
            out_ref[0] = acc0_ref[:, :]
            out_ref[1] = acc1_ref[:, :]
    return pl.pallas_call(
        body,
        grid=(nch,),
        in_specs=[pl.BlockSpec((K,), lambda j: (j,),
                               memory_space=pltpu.SMEM),
                  pl.BlockSpec((K,), lambda j: (j,),
                               memory_space=pltpu.SMEM),
                  pl.BlockSpec((2, npad, d), lambda j: (0, 0, 0))],
        out_specs=pl.BlockSpec((2, npad, d), lambda j: (0, 0, 0)),
        out_shape=jax.ShapeDtypeStruct((2, npad, d), jnp.float32),
        scratch_shapes=[pltpu.VMEM((npad, d), jnp.float32),
                        pltpu.VMEM((npad, d), jnp.float32)],
    )(srcb, dstb, tables)


def _tc_scale(degp, h1, h1perm, npad, d, blk=512):
    def body(dinv_ref, h1_ref, hp_ref, t1_ref):
        dv = dinv_ref[:, :]
        t1_ref[0, :, :] = dv * h1_ref[:, :]
        t1_ref[1, :, :] = dv * hp_ref[:, :]
    return pl.pallas_call(
        body,
        grid=(npad // blk,),
        in_specs=[pl.BlockSpec((blk, d), lambda i: (i, 0)),
                  pl.BlockSpec((blk, d), lambda i: (i, 0)),
                  pl.BlockSpec((blk, d), lambda i: (i, 0))],
        out_specs=pl.BlockSpec((2, blk, d), lambda i: (0, i, 0)),
        out_shape=jax.ShapeDtypeStruct((2, npad, d), jnp.float32),
    )(degp, h1, h1perm)


def _tc_l1finish(acc1, dinv, b1, w2, n, npad, d, blk=512):
    def body(acc_ref, dinv_ref, b1_ref, w2_ref, t2_ref):
        i = pl.program_id(0)
        rows = lax.broadcasted_iota(jnp.int32, (blk, d), 0) + i * blk
        m = rows < n
        dv = dinv_ref[:, :]
        for cc in range(2):
            z = jnp.maximum(acc_ref[cc] * dv + b1_ref[:, :], 0.0)
            z = jnp.where(m, z, 0.0)
            t2_ref[cc, :, :] = jnp.dot(z, w2_ref[:, :],
                                       preferred_element_type=jnp.float32) * dv
    return pl.pallas_call(
        body,
        grid=(npad // blk,),
        in_specs=[pl.BlockSpec((2, blk, d), lambda i: (0, i, 0)),
                  pl.BlockSpec((blk, d), lambda i: (i, 0)),
                  pl.BlockSpec((1, d), lambda i: (0, 0)),
                  pl.BlockSpec((d, d), lambda i: (0, 0))],
        out_specs=pl.BlockSpec((2, blk, d), lambda i: (0, i, 0)),
        out_shape=jax.ShapeDtypeStruct((2, npad, d), jnp.float32),
    )(acc1, dinv, b1, w2)


def _tc_proj(acc2, dinv, b2, wd, n, npad, d, blk=512):
    def body(acc_ref, dinv_ref, b2_ref, wd_ref, proj_ref, sum_ref):
        i = pl.program_id(0)

        @pl.when(i == 0)
        def _():
            sum_ref[:, :] = jnp.zeros_like(sum_ref)

        rows = lax.broadcasted_iota(jnp.int32, (blk, d), 0) + i * blk
        z = acc_ref[0] * dinv_ref[:, :] + b2_ref[:, :]
        z = jnp.where(rows < n, z, 0.0)
        sum_ref[:, :] += jnp.sum(z, axis=0, keepdims=True)

        @pl.when(i == pl.num_programs(0) - 1)
        def _():
            summ = jax.nn.sigmoid(sum_ref[:, :] / float(n))
            proj_ref[:, :] = lax.dot_general(
                summ, wd_ref[:, :], (((1,), (1,)), ((), ())),
                preferred_element_type=jnp.float32)
    return pl.pallas_call(
        body,
        grid=(npad // blk,),
        in_specs=[pl.BlockSpec((1, blk, d), lambda i: (0, i, 0)),
                  pl.BlockSpec((blk, d), lambda i: (i, 0)),
                  pl.BlockSpec((1, d), lambda i: (0, 0)),
                  pl.BlockSpec((d, d), lambda i: (0, 0))],
        out_specs=pl.BlockSpec((1, d), lambda i: (0, 0)),
        out_shape=jax.ShapeDtypeStruct((1, d), jnp.float32),
        scratch_shapes=[pltpu.VMEM((1, d), jnp.float32)],
    )(acc2, dinv, b2, wd)


def _tc_loss(acc2, dinv, b2, proj, n, npad, d, blk=512):
    def body(acc_ref, dinv_ref, b2_ref, proj_ref, out_ref, s_ref):
        i = pl.program_id(0)

        @pl.when(i == 0)
        def _():
            s_ref[0] = 0.0
            s_ref[1] = 0.0

        rows = lax.broadcasted_iota(jnp.int32, (blk, 1), 0) + i * blk
        m = rows < n
        dv = dinv_ref[:, :]
        dn = (((1,), (1,)), ((), ()))
        pz = acc_ref[0] * dv + b2_ref[:, :]
        nz = acc_ref[1] * dv + b2_ref[:, :]
        sp = lax.dot_general(pz, proj_ref[:, :], dn,
                             preferred_element_type=jnp.float32)
        sn = lax.dot_general(nz, proj_ref[:, :], dn,
                             preferred_element_type=jnp.float32)
        lp = jnp.where(m, jnp.log(jax.nn.sigmoid(sp) + EPS), 0.0)
        ln = jnp.where(m, jnp.log(1.0 - jax.nn.sigmoid(sn) + EPS), 0.0)
        s_ref[0] += jnp.sum(lp)
        s_ref[1] += jnp.sum(ln)

        @pl.when(i == pl.num_programs(0) - 1)
        def _():
            out_ref[:, :] = jnp.full((1, 1), -(s_ref[0] + s_ref[1]) / float(n),
                                     jnp.float32)
    return pl.pallas_call(
        body,
        grid=(npad // blk,),
        in_specs=[pl.BlockSpec((2, blk, d), lambda i: (0, i, 0)),
                  pl.BlockSpec((blk, d), lambda i: (i, 0)),
                  pl.BlockSpec((1, d), lambda i: (0, 0)),
                  pl.BlockSpec((1, d), lambda i: (0, 0))],
        out_specs=pl.BlockSpec((1, 1), lambda i: (0, 0)),
        out_shape=jax.ShapeDtypeStruct((1, 1), jnp.float32),
        scratch_shapes=[pltpu.SMEM((2,), jnp.float32)],
    )(acc2, dinv, b2, proj)


def kernel(x, edge_index, perm, W1, b1, W2, b2, Wd):
    n, d = x.shape
    e = edge_index.shape[1]
    npad = ((n + 1 + 511) // 512) * 512

    src = edge_index[0].astype(jnp.int32)
    dst = edge_index[1].astype(jnp.int32)

    K = 16384
    ne = ((e + K - 1) // K) * K
    src_p = jnp.pad(src, (0, ne - e), constant_values=n)
    dst_p = jnp.pad(dst, (0, ne - e), constant_values=n)

    perm_p = jnp.pad(perm.astype(jnp.int32), (0, npad - n),
                     constant_values=n)
    x_p = jnp.pad(x, ((0, npad - n), (0, 0)))
    b1r = b1.reshape(1, d)
    b2r = b2.reshape(1, d)

    h1 = _tc_mm(x_p, W1, npad, d)
    dinv = _tc_deg(dst_p, npad, d, ne, K)
    h1perm = _sc_gather(h1, perm_p, npad, d)
    T1 = _tc_scale(dinv, h1, h1perm, npad, d)
    acc1 = _tc_scatter(T1, src_p, dst_p, npad, d, ne, K)
    T2 = _tc_l1finish(acc1, dinv, b1r, W2, n, npad, d)
    acc2 = _tc_scatter(T2, src_p, dst_p, npad, d, ne, K)
    proj = _tc_proj(acc2, dinv, b2r, Wd, n, npad, d)
    out = _tc_loss(acc2, dinv, b2r, proj, n, npad, d)
    return out.reshape(())


# speedup vs baseline: 5.0840x; 1.4847x over previous
import functools

import jax
import jax.numpy as jnp
from jax import lax
from jax.experimental import pallas as pl
from jax.experimental.pallas import tpu as pltpu
from jax.experimental.pallas import tpu_sc as plsc

EPS = 1e-15

NC = 2
NT = 16
CHUNK = 128


def _mesh():
    return plsc.VectorSubcoreMesh(core_axis_name="c", subcore_axis_name="s")


def _sc_gather(table, idx_flat, npad, d):
    rows_per_w = npad // (NC * NT)
    nch = rows_per_w // 64

    @functools.partial(
        pl.kernel, mesh=_mesh(),
        out_type=jax.ShapeDtypeStruct((npad, d), jnp.float32),
        scratch_types=[
            pltpu.VMEM((64,), jnp.int32),
            pltpu.VMEM((64, d), jnp.float32),
            pltpu.SemaphoreType.DMA,
        ],
    )
    def k(tab_hbm, idx_hbm, out_hbm, idx_v, rows_v, sem):
        c = lax.axis_index("c")
        s = lax.axis_index("s")
        wid = c * NT + s

        def body(j, carry):
            pltpu.sync_copy(idx_hbm.at[pl.ds(wid * rows_per_w + j * 64, 64)], idx_v)
            pltpu.async_copy(tab_hbm.at[idx_v], rows_v, sem).wait()
            pltpu.sync_copy(rows_v, out_hbm.at[pl.ds(wid * rows_per_w + j * 64, 64)])
            return carry
        lax.fori_loop(0, nch, body, 0)

    return k(table, idx_flat)


def _tc_mm(x, w, npad, d, blk=1024):
    def body(x_ref, w_ref, o_ref):
        o_ref[:, :] = jnp.dot(x_ref[:, :], w_ref[:, :],
                              preferred_element_type=jnp.float32)
    return pl.pallas_call(
        body,
        grid=(npad // blk,),
        in_specs=[pl.BlockSpec((blk, d), lambda i: (i, 0)),
                  pl.BlockSpec((d, d), lambda i: (0, 0))],
        out_specs=pl.BlockSpec((blk, d), lambda i: (i, 0)),
        out_shape=jax.ShapeDtypeStruct((npad, d), jnp.float32),
    )(x, w)


def _tc_deg(dstb, npad, d, ne, K):
    nch = ne // K

    def body(dst_ref, out_ref, acc_ref, acc2_ref):
        j = pl.program_id(0)

        @pl.when(j == 0)
        def _():
            acc_ref[:, :] = jnp.zeros_like(acc_ref)
            acc2_ref[:, :] = jnp.zeros_like(acc2_ref)

        one_row = jnp.full((1, d), 1.0, jnp.float32)

        def eb(i, carry):
            d0 = dst_ref[i]
            d1 = dst_ref[i + K // 2]
            acc_ref[pl.ds(d0, 1), :] += one_row
            acc2_ref[pl.ds(d1, 1), :] += one_row
            return carry
        lax.fori_loop(0, K // 2, eb, 0, unroll=8)

        @pl.when(j == nch - 1)
        def _():
            out_ref[:, :] = lax.rsqrt(acc_ref[:, :] + acc2_ref[:, :] + 1.0)
    return pl.pallas_call(
        body,
        grid=(nch,),
        in_specs=[pl.BlockSpec((K,), lambda j: (j,),
                               memory_space=pltpu.SMEM)],
        out_specs=pl.BlockSpec((npad, d), lambda j: (0, 0)),
        out_shape=jax.ShapeDtypeStruct((npad, d), jnp.float32),
        scratch_shapes=[pltpu.VMEM((npad, d), jnp.float32),
                        pltpu.VMEM((npad, d), jnp.float32)],
    )(dstb)


def _tc_scatter(tables, srcb, dstb, npad, d, ne, K):
    nch = ne // K

    def body(src_ref, dst_ref, tab_ref, out_ref,
             acc0_ref, acc1_ref, acc2_ref, acc3_ref):
        j = pl.program_id(0)

        @pl.when(j == 0)
        def _():
            acc0_ref[:, :] = tab_ref[0]
            acc1_ref[:, :] = tab_ref[1]
            acc2_ref[:, :] = jnp.zeros_like(acc2_ref)
            acc3_ref[:, :] = jnp.zeros_like(acc3_ref)

        def eb(i, carry):
            s0 = src_ref[i]
            d0 = dst_ref[i]
            s1 = src_ref[i + K // 2]
            d1 = dst_ref[i + K // 2]
            acc0_ref[pl.ds(d0, 1), :] += tab_ref[0, pl.ds(s0, 1), :]
            acc1_ref[pl.ds(d0, 1), :] += tab_ref[1, pl.ds(s0, 1), :]
            acc2_ref[pl.ds(d1, 1), :] += tab_ref[0, pl.ds(s1, 1), :]
            acc3_ref[pl.ds(d1, 1), :] += tab_ref[1, pl.ds(s1, 1), :]
            return carry
        lax.fori_loop(0, K // 2, eb, 0, unroll=8)

        @pl.when(j == nch - 1)
        def _():
            out_ref[0] = acc0_ref[:, :] + acc2_ref[:, :]
            out_ref[1] = acc1_ref[:, :] + acc3_ref[:, :]
    return pl.pallas_call(
        body,
        grid=(nch,),
        in_specs=[pl.BlockSpec((K,), lambda j: (j,),
                               memory_space=pltpu.SMEM),
                  pl.BlockSpec((K,), lambda j: (j,),
                               memory_space=pltpu.SMEM),
                  pl.BlockSpec((2, npad, d), lambda j: (0, 0, 0))],
        out_specs=pl.BlockSpec((2, npad, d), lambda j: (0, 0, 0)),
        out_shape=jax.ShapeDtypeStruct((2, npad, d), jnp.float32),
        scratch_shapes=[pltpu.VMEM((npad, d), jnp.float32),
                        pltpu.VMEM((npad, d), jnp.float32),
                        pltpu.VMEM((npad, d), jnp.float32),
                        pltpu.VMEM((npad, d), jnp.float32)],
    )(srcb, dstb, tables)


def _tc_scale(degp, h1, h1perm, npad, d, blk=512):
    def body(dinv_ref, h1_ref, hp_ref, t1_ref):
        dv = dinv_ref[:, :]
        t1_ref[0, :, :] = dv * h1_ref[:, :]
        t1_ref[1, :, :] = dv * hp_ref[:, :]
    return pl.pallas_call(
        body,
        grid=(npad // blk,),
        in_specs=[pl.BlockSpec((blk, d), lambda i: (i, 0)),
                  pl.BlockSpec((blk, d), lambda i: (i, 0)),
                  pl.BlockSpec((blk, d), lambda i: (i, 0))],
        out_specs=pl.BlockSpec((2, blk, d), lambda i: (0, i, 0)),
        out_shape=jax.ShapeDtypeStruct((2, npad, d), jnp.float32),
    )(degp, h1, h1perm)


def _tc_l1finish(acc1, dinv, b1, w2, n, npad, d, blk=512):
    def body(acc_ref, dinv_ref, b1_ref, w2_ref, t2_ref):
        i = pl.program_id(0)
        rows = lax.broadcasted_iota(jnp.int32, (blk, d), 0) + i * blk
        m = rows < n
        dv = dinv_ref[:, :]
        for cc in range(2):
            z = jnp.maximum(acc_ref[cc] * dv + b1_ref[:, :], 0.0)
            z = jnp.where(m, z, 0.0)
            t2_ref[cc, :, :] = jnp.dot(z, w2_ref[:, :],
                                       preferred_element_type=jnp.float32) * dv
    return pl.pallas_call(
        body,
        grid=(npad // blk,),
        in_specs=[pl.BlockSpec((2, blk, d), lambda i: (0, i, 0)),
                  pl.BlockSpec((blk, d), lambda i: (i, 0)),
                  pl.BlockSpec((1, d), lambda i: (0, 0)),
                  pl.BlockSpec((d, d), lambda i: (0, 0))],
        out_specs=pl.BlockSpec((2, blk, d), lambda i: (0, i, 0)),
        out_shape=jax.ShapeDtypeStruct((2, npad, d), jnp.float32),
    )(acc1, dinv, b1, w2)


def _tc_proj(acc2, dinv, b2, wd, n, npad, d, blk=512):
    def body(acc_ref, dinv_ref, b2_ref, wd_ref, proj_ref, sum_ref):
        i = pl.program_id(0)

        @pl.when(i == 0)
        def _():
            sum_ref[:, :] = jnp.zeros_like(sum_ref)

        rows = lax.broadcasted_iota(jnp.int32, (blk, d), 0) + i * blk
        z = acc_ref[0] * dinv_ref[:, :] + b2_ref[:, :]
        z = jnp.where(rows < n, z, 0.0)
        sum_ref[:, :] += jnp.sum(z, axis=0, keepdims=True)

        @pl.when(i == pl.num_programs(0) - 1)
        def _():
            summ = jax.nn.sigmoid(sum_ref[:, :] / float(n))
            proj_ref[:, :] = lax.dot_general(
                summ, wd_ref[:, :], (((1,), (1,)), ((), ())),
                preferred_element_type=jnp.float32)
    return pl.pallas_call(
        body,
        grid=(npad // blk,),
        in_specs=[pl.BlockSpec((1, blk, d), lambda i: (0, i, 0)),
                  pl.BlockSpec((blk, d), lambda i: (i, 0)),
                  pl.BlockSpec((1, d), lambda i: (0, 0)),
                  pl.BlockSpec((d, d), lambda i: (0, 0))],
        out_specs=pl.BlockSpec((1, d), lambda i: (0, 0)),
        out_shape=jax.ShapeDtypeStruct((1, d), jnp.float32),
        scratch_shapes=[pltpu.VMEM((1, d), jnp.float32)],
    )(acc2, dinv, b2, wd)


def _tc_loss(acc2, dinv, b2, proj, n, npad, d, blk=512):
    def body(acc_ref, dinv_ref, b2_ref, proj_ref, out_ref, s_ref):
        i = pl.program_id(0)

        @pl.when(i == 0)
        def _():
            s_ref[0] = 0.0
            s_ref[1] = 0.0

        rows = lax.broadcasted_iota(jnp.int32, (blk, 1), 0) + i * blk
        m = rows < n
        dv = dinv_ref[:, :]
        dn = (((1,), (1,)), ((), ()))
        pz = acc_ref[0] * dv + b2_ref[:, :]
        nz = acc_ref[1] * dv + b2_ref[:, :]
        sp = lax.dot_general(pz, proj_ref[:, :], dn,
                             preferred_element_type=jnp.float32)
        sn = lax.dot_general(nz, proj_ref[:, :], dn,
                             preferred_element_type=jnp.float32)
        lp = jnp.where(m, jnp.log(jax.nn.sigmoid(sp) + EPS), 0.0)
        ln = jnp.where(m, jnp.log(1.0 - jax.nn.sigmoid(sn) + EPS), 0.0)
        s_ref[0] += jnp.sum(lp)
        s_ref[1] += jnp.sum(ln)

        @pl.when(i == pl.num_programs(0) - 1)
        def _():
            out_ref[:, :] = jnp.full((1, 1), -(s_ref[0] + s_ref[1]) / float(n),
                                     jnp.float32)
    return pl.pallas_call(
        body,
        grid=(npad // blk,),
        in_specs=[pl.BlockSpec((2, blk, d), lambda i: (0, i, 0)),
                  pl.BlockSpec((blk, d), lambda i: (i, 0)),
                  pl.BlockSpec((1, d), lambda i: (0, 0)),
                  pl.BlockSpec((1, d), lambda i: (0, 0))],
        out_specs=pl.BlockSpec((1, 1), lambda i: (0, 0)),
        out_shape=jax.ShapeDtypeStruct((1, 1), jnp.float32),
        scratch_shapes=[pltpu.SMEM((2,), jnp.float32)],
    )(acc2, dinv, b2, proj)


def kernel(x, edge_index, perm, W1, b1, W2, b2, Wd):
    n, d = x.shape
    e = edge_index.shape[1]
    npad = ((n + 1 + 511) // 512) * 512

    src = edge_index[0].astype(jnp.int32)
    dst = edge_index[1].astype(jnp.int32)

    K = 16384
    ne = ((e + K - 1) // K) * K
    src_p = jnp.pad(src, (0, ne - e), constant_values=n)
    dst_p = jnp.pad(dst, (0, ne - e), constant_values=n)

    perm_p = jnp.pad(perm.astype(jnp.int32), (0, npad - n),
                     constant_values=n)
    x_p = jnp.pad(x, ((0, npad - n), (0, 0)))
    b1r = b1.reshape(1, d)
    b2r = b2.reshape(1, d)

    h1 = _tc_mm(x_p, W1, npad, d)
    dinv = _tc_deg(dst_p, npad, d, ne, K)
    h1perm = _sc_gather(h1, perm_p, npad, d)
    T1 = _tc_scale(dinv, h1, h1perm, npad, d)
    acc1 = _tc_scatter(T1, src_p, dst_p, npad, d, ne, K)
    T2 = _tc_l1finish(acc1, dinv, b1r, W2, n, npad, d)
    acc2 = _tc_scatter(T2, src_p, dst_p, npad, d, ne, K)
    proj = _tc_proj(acc2, dinv, b2r, Wd, n, npad, d)
    out = _tc_loss(acc2, dinv, b2r, proj, n, npad, d)
    return out.reshape(())


# speedup vs baseline: 6.6453x; 1.3071x over previous
import functools

import jax
import jax.numpy as jnp
from jax import lax
from jax.experimental import pallas as pl
from jax.experimental.pallas import tpu as pltpu
from jax.experimental.pallas import tpu_sc as plsc

EPS = 1e-15

NC = 2
NT = 16
CHUNK = 128


def _mesh():
    return plsc.VectorSubcoreMesh(core_axis_name="c", subcore_axis_name="s")


def _sc_gather(table, idx_flat, npad, d):
    rows_per_w = npad // (NC * NT)
    nch = rows_per_w // 64

    @functools.partial(
        pl.kernel, mesh=_mesh(),
        out_type=jax.ShapeDtypeStruct((npad, d), jnp.float32),
        scratch_types=[
            pltpu.VMEM((64,), jnp.int32),
            pltpu.VMEM((64, d), jnp.float32),
            pltpu.SemaphoreType.DMA,
        ],
    )
    def k(tab_hbm, idx_hbm, out_hbm, idx_v, rows_v, sem):
        c = lax.axis_index("c")
        s = lax.axis_index("s")
        wid = c * NT + s

        def body(j, carry):
            pltpu.sync_copy(idx_hbm.at[pl.ds(wid * rows_per_w + j * 64, 64)], idx_v)
            pltpu.async_copy(tab_hbm.at[idx_v], rows_v, sem).wait()
            pltpu.sync_copy(rows_v, out_hbm.at[pl.ds(wid * rows_per_w + j * 64, 64)])
            return carry
        lax.fori_loop(0, nch, body, 0)

    return k(table, idx_flat)


def _tc_mm(x, w, npad, d, blk=1024):
    def body(x_ref, w_ref, o_ref):
        o_ref[:, :] = jnp.dot(x_ref[:, :], w_ref[:, :],
                              preferred_element_type=jnp.float32)
    return pl.pallas_call(
        body,
        grid=(npad // blk,),
        in_specs=[pl.BlockSpec((blk, d), lambda i: (i, 0)),
                  pl.BlockSpec((d, d), lambda i: (0, 0))],
        out_specs=pl.BlockSpec((blk, d), lambda i: (i, 0)),
        out_shape=jax.ShapeDtypeStruct((npad, d), jnp.float32),
    )(x, w)


def _tc_deg(dstb, npad, d, ne, K):
    nch = ne // K

    def body(dst_ref, out_ref, *accs):
        j = pl.program_id(0)

        @pl.when(j == 0)
        def _():
            for a in accs:
                a[:, :] = jnp.zeros_like(a)

        one_row = jnp.full((1, d), 1.0, jnp.float32)
        nw = len(accs)

        def eb(i, carry):
            for w, a in enumerate(accs):
                dw = dst_ref[i + w * (K // nw)]
                a[pl.ds(dw, 1), :] += one_row
            return carry
        lax.fori_loop(0, K // nw, eb, 0, unroll=8)

        @pl.when(j == nch - 1)
        def _():
            tot = accs[0][:, :]
            for a in accs[1:]:
                tot = tot + a[:, :]
            out_ref[:, :] = lax.rsqrt(tot + 1.0)
    return pl.pallas_call(
        body,
        grid=(nch,),
        in_specs=[pl.BlockSpec((K,), lambda j: (j,),
                               memory_space=pltpu.SMEM)],
        out_specs=pl.BlockSpec((npad, d), lambda j: (0, 0)),
        out_shape=jax.ShapeDtypeStruct((npad, d), jnp.float32),
        scratch_shapes=[pltpu.VMEM((npad, d), jnp.float32)] * 4,
    )(dstb)


def _tc_scatter(tables, srcb, dstb, npad, d, ne, K):
    nch = ne // K

    def body(src_ref, dst_ref, tab_ref, out_ref, *accs):
        j = pl.program_id(0)
        nw = len(accs) // 2 + 1

        @pl.when(j == 0)
        def _():
            out_ref[0] = tab_ref[0]
            out_ref[1] = tab_ref[1]
            for a in accs:
                a[:, :] = jnp.zeros_like(a)

        def eb(i, carry):
            for w in range(nw):
                sw = src_ref[i + w * (K // nw)]
                dw = dst_ref[i + w * (K // nw)]
                if w == 0:
                    out_ref[0, pl.ds(dw, 1), :] += tab_ref[0, pl.ds(sw, 1), :]
                    out_ref[1, pl.ds(dw, 1), :] += tab_ref[1, pl.ds(sw, 1), :]
                else:
                    accs[2 * (w - 1)][pl.ds(dw, 1), :] += \
                        tab_ref[0, pl.ds(sw, 1), :]
                    accs[2 * (w - 1) + 1][pl.ds(dw, 1), :] += \
                        tab_ref[1, pl.ds(sw, 1), :]
            return carry
        lax.fori_loop(0, K // nw, eb, 0, unroll=8)

        @pl.when(j == nch - 1)
        def _():
            for c in range(2):
                tot = out_ref[c]
                for w in range(1, nw):
                    tot = tot + accs[2 * (w - 1) + c][:, :]
                out_ref[c] = tot
    return pl.pallas_call(
        body,
        grid=(nch,),
        in_specs=[pl.BlockSpec((K,), lambda j: (j,),
                               memory_space=pltpu.SMEM),
                  pl.BlockSpec((K,), lambda j: (j,),
                               memory_space=pltpu.SMEM),
                  pl.BlockSpec((2, npad, d), lambda j: (0, 0, 0))],
        out_specs=pl.BlockSpec((2, npad, d), lambda j: (0, 0, 0)),
        out_shape=jax.ShapeDtypeStruct((2, npad, d), jnp.float32),
        scratch_shapes=[pltpu.VMEM((npad, d), jnp.float32)] * 6,
    )(srcb, dstb, tables)


def _tc_scale(degp, h1, h1perm, npad, d, blk=512):
    def body(dinv_ref, h1_ref, hp_ref, t1_ref):
        dv = dinv_ref[:, :]
        t1_ref[0, :, :] = dv * h1_ref[:, :]
        t1_ref[1, :, :] = dv * hp_ref[:, :]
    return pl.pallas_call(
        body,
        grid=(npad // blk,),
        in_specs=[pl.BlockSpec((blk, d), lambda i: (i, 0)),
                  pl.BlockSpec((blk, d), lambda i: (i, 0)),
                  pl.BlockSpec((blk, d), lambda i: (i, 0))],
        out_specs=pl.BlockSpec((2, blk, d), lambda i: (0, i, 0)),
        out_shape=jax.ShapeDtypeStruct((2, npad, d), jnp.float32),
    )(degp, h1, h1perm)


def _tc_l1finish(acc1, dinv, b1, w2, n, npad, d, blk=512):
    def body(acc_ref, dinv_ref, b1_ref, w2_ref, t2_ref):
        i = pl.program_id(0)
        rows = lax.broadcasted_iota(jnp.int32, (blk, d), 0) + i * blk
        m = rows < n
        dv = dinv_ref[:, :]
        for cc in range(2):
            z = jnp.maximum(acc_ref[cc] * dv + b1_ref[:, :], 0.0)
            z = jnp.where(m, z, 0.0)
            t2_ref[cc, :, :] = jnp.dot(z, w2_ref[:, :],
                                       preferred_element_type=jnp.float32) * dv
    return pl.pallas_call(
        body,
        grid=(npad // blk,),
        in_specs=[pl.BlockSpec((2, blk, d), lambda i: (0, i, 0)),
                  pl.BlockSpec((blk, d), lambda i: (i, 0)),
                  pl.BlockSpec((1, d), lambda i: (0, 0)),
                  pl.BlockSpec((d, d), lambda i: (0, 0))],
        out_specs=pl.BlockSpec((2, blk, d), lambda i: (0, i, 0)),
        out_shape=jax.ShapeDtypeStruct((2, npad, d), jnp.float32),
    )(acc1, dinv, b1, w2)


def _tc_proj(acc2, dinv, b2, wd, n, npad, d, blk=512):
    def body(acc_ref, dinv_ref, b2_ref, wd_ref, proj_ref, sum_ref):
        i = pl.program_id(0)

        @pl.when(i == 0)
        def _():
            sum_ref[:, :] = jnp.zeros_like(sum_ref)

        rows = lax.broadcasted_iota(jnp.int32, (blk, d), 0) + i * blk
        z = acc_ref[0] * dinv_ref[:, :] + b2_ref[:, :]
        z = jnp.where(rows < n, z, 0.0)
        sum_ref[:, :] += jnp.sum(z, axis=0, keepdims=True)

        @pl.when(i == pl.num_programs(0) - 1)
        def _():
            summ = jax.nn.sigmoid(sum_ref[:, :] / float(n))
            proj_ref[:, :] = lax.dot_general(
                summ, wd_ref[:, :], (((1,), (1,)), ((), ())),
                preferred_element_type=jnp.float32)
    return pl.pallas_call(
        body,
        grid=(npad // blk,),
        in_specs=[pl.BlockSpec((1, blk, d), lambda i: (0, i, 0)),
                  pl.BlockSpec((blk, d), lambda i: (i, 0)),
                  pl.BlockSpec((1, d), lambda i: (0, 0)),
                  pl.BlockSpec((d, d), lambda i: (0, 0))],
        out_specs=pl.BlockSpec((1, d), lambda i: (0, 0)),
        out_shape=jax.ShapeDtypeStruct((1, d), jnp.float32),
        scratch_shapes=[pltpu.VMEM((1, d), jnp.float32)],
    )(acc2, dinv, b2, wd)


def _tc_loss(acc2, dinv, b2, proj, n, npad, d, blk=512):
    def body(acc_ref, dinv_ref, b2_ref, proj_ref, out_ref, s_ref):
        i = pl.program_id(0)

        @pl.when(i == 0)
        def _():
            s_ref[0] = 0.0
            s_ref[1] = 0.0

        rows = lax.broadcasted_iota(jnp.int32, (blk, 1), 0) + i * blk
        m = rows < n
        dv = dinv_ref[:, :]
        dn = (((1,), (1,)), ((), ()))
        pz = acc_ref[0] * dv + b2_ref[:, :]
        nz = acc_ref[1] * dv + b2_ref[:, :]
        sp = lax.dot_general(pz, proj_ref[:, :], dn,
                             preferred_element_type=jnp.float32)
        sn = lax.dot_general(nz, proj_ref[:, :], dn,
                             preferred_element_type=jnp.float32)
        lp = jnp.where(m, jnp.log(jax.nn.sigmoid(sp) + EPS), 0.0)
        ln = jnp.where(m, jnp.log(1.0 - jax.nn.sigmoid(sn) + EPS), 0.0)
        s_ref[0] += jnp.sum(lp)
        s_ref[1] += jnp.sum(ln)

        @pl.when(i == pl.num_programs(0) - 1)
        def _():
            out_ref[:, :] = jnp.full((1, 1), -(s_ref[0] + s_ref[1]) / float(n),
                                     jnp.float32)
    return pl.pallas_call(
        body,
        grid=(npad // blk,),
        in_specs=[pl.BlockSpec((2, blk, d), lambda i: (0, i, 0)),
                  pl.BlockSpec((blk, d), lambda i: (i, 0)),
                  pl.BlockSpec((1, d), lambda i: (0, 0)),
                  pl.BlockSpec((1, d), lambda i: (0, 0))],
        out_specs=pl.BlockSpec((1, 1), lambda i: (0, 0)),
        out_shape=jax.ShapeDtypeStruct((1, 1), jnp.float32),
        scratch_shapes=[pltpu.SMEM((2,), jnp.float32)],
    )(acc2, dinv, b2, proj)


def kernel(x, edge_index, perm, W1, b1, W2, b2, Wd):
    n, d = x.shape
    e = edge_index.shape[1]
    npad = ((n + 1 + 511) // 512) * 512

    src = edge_index[0].astype(jnp.int32)
    dst = edge_index[1].astype(jnp.int32)

    K = 16384
    ne = ((e + K - 1) // K) * K
    src_p = jnp.pad(src, (0, ne - e), constant_values=n)
    dst_p = jnp.pad(dst, (0, ne - e), constant_values=n)

    perm_p = jnp.pad(perm.astype(jnp.int32), (0, npad - n),
                     constant_values=n)
    x_p = jnp.pad(x, ((0, npad - n), (0, 0)))
    b1r = b1.reshape(1, d)
    b2r = b2.reshape(1, d)

    h1 = _tc_mm(x_p, W1, npad, d)
    dinv = _tc_deg(dst_p, npad, d, ne, K)
    h1perm = _sc_gather(h1, perm_p, npad, d)
    T1 = _tc_scale(dinv, h1, h1perm, npad, d)
    acc1 = _tc_scatter(T1, src_p, dst_p, npad, d, ne, K)
    T2 = _tc_l1finish(acc1, dinv, b1r, W2, n, npad, d)
    acc2 = _tc_scatter(T2, src_p, dst_p, npad, d, ne, K)
    proj = _tc_proj(acc2, dinv, b2r, Wd, n, npad, d)
    out = _tc_loss(acc2, dinv, b2r, proj, n, npad, d)
    return out.reshape(())


# speedup vs baseline: 7.0803x; 1.0655x over previous
import functools

import jax
import jax.numpy as jnp
from jax import lax
from jax.experimental import pallas as pl
from jax.experimental.pallas import tpu as pltpu
from jax.experimental.pallas import tpu_sc as plsc

EPS = 1e-15

NC = 2
NT = 16
CHUNK = 128


def _mesh():
    return plsc.VectorSubcoreMesh(core_axis_name="c", subcore_axis_name="s")


def _sc_gather(table, idx_flat, npad, d):
    rows_per_w = npad // (NC * NT)
    nch = rows_per_w // 64

    @functools.partial(
        pl.kernel, mesh=_mesh(),
        out_type=jax.ShapeDtypeStruct((npad, d), jnp.float32),
        scratch_types=[
            pltpu.VMEM((64,), jnp.int32),
            pltpu.VMEM((64, d), jnp.float32),
            pltpu.SemaphoreType.DMA,
        ],
    )
    def k(tab_hbm, idx_hbm, out_hbm, idx_v, rows_v, sem):
        c = lax.axis_index("c")
        s = lax.axis_index("s")
        wid = c * NT + s

        def body(j, carry):
            pltpu.sync_copy(idx_hbm.at[pl.ds(wid * rows_per_w + j * 64, 64)], idx_v)
            pltpu.async_copy(tab_hbm.at[idx_v], rows_v, sem).wait()
            pltpu.sync_copy(rows_v, out_hbm.at[pl.ds(wid * rows_per_w + j * 64, 64)])
            return carry
        lax.fori_loop(0, nch, body, 0)

    return k(table, idx_flat)


def _tc_mm(x, w, npad, d, blk=1024):
    def body(x_ref, w_ref, o_ref):
        o_ref[:, :] = jnp.dot(x_ref[:, :], w_ref[:, :],
                              preferred_element_type=jnp.float32)
    return pl.pallas_call(
        body,
        grid=(npad // blk,),
        in_specs=[pl.BlockSpec((blk, d), lambda i: (i, 0)),
                  pl.BlockSpec((d, d), lambda i: (0, 0))],
        out_specs=pl.BlockSpec((blk, d), lambda i: (i, 0)),
        out_shape=jax.ShapeDtypeStruct((npad, d), jnp.float32),
    )(x, w)


def _tc_deg(dstb, npad, d, ne, K):
    nch = ne // K

    def body(dst_ref, out_ref, *accs):
        j = pl.program_id(0)

        @pl.when(j == 0)
        def _():
            for a in accs:
                a[:, :] = jnp.zeros_like(a)

        one_row = jnp.full((1, d), 1.0, jnp.float32)
        nw = len(accs)

        def eb(i, carry):
            for w, a in enumerate(accs):
                dw = dst_ref[i + w * (K // nw)]
                a[pl.ds(dw, 1), :] += one_row
            return carry
        lax.fori_loop(0, K // nw, eb, 0, unroll=8)

        @pl.when(j == nch - 1)
        def _():
            tot = accs[0][:, :]
            for a in accs[1:]:
                tot = tot + a[:, :]
            out_ref[:, :] = lax.rsqrt(tot + 1.0)
    return pl.pallas_call(
        body,
        grid=(nch,),
        in_specs=[pl.BlockSpec((K,), lambda j: (j,),
                               memory_space=pltpu.SMEM)],
        out_specs=pl.BlockSpec((npad, d), lambda j: (0, 0)),
        out_shape=jax.ShapeDtypeStruct((npad, d), jnp.float32),
        scratch_shapes=[pltpu.VMEM((npad, d), jnp.float32)] * 8,
    )(dstb)


def _tc_scatter(tables, srcb, dstb, npad, d, ne, K):
    nch = ne // K

    def body(src_ref, dst_ref, tab_ref, out_ref, *accs):
        j = pl.program_id(0)
        nw = len(accs) // 2 + 1

        @pl.when(j == 0)
        def _():
            out_ref[0] = tab_ref[0]
            out_ref[1] = tab_ref[1]
            for a in accs:
                a[:, :] = jnp.zeros_like(a)

        def eb(i, carry):
            for w in range(nw):
                sw = src_ref[i + w * (K // nw)]
                dw = dst_ref[i + w * (K // nw)]
                if w == 0:
                    out_ref[0, pl.ds(dw, 1), :] += tab_ref[0, pl.ds(sw, 1), :]
                    out_ref[1, pl.ds(dw, 1), :] += tab_ref[1, pl.ds(sw, 1), :]
                else:
                    accs[2 * (w - 1)][pl.ds(dw, 1), :] += \
                        tab_ref[0, pl.ds(sw, 1), :]
                    accs[2 * (w - 1) + 1][pl.ds(dw, 1), :] += \
                        tab_ref[1, pl.ds(sw, 1), :]
            return carry
        lax.fori_loop(0, K // nw, eb, 0, unroll=8)

        @pl.when(j == nch - 1)
        def _():
            for c in range(2):
                tot = out_ref[c]
                for w in range(1, nw):
                    tot = tot + accs[2 * (w - 1) + c][:, :]
                out_ref[c] = tot
    return pl.pallas_call(
        body,
        grid=(nch,),
        in_specs=[pl.BlockSpec((K,), lambda j: (j,),
                               memory_space=pltpu.SMEM),
                  pl.BlockSpec((K,), lambda j: (j,),
                               memory_space=pltpu.SMEM),
                  pl.BlockSpec((2, npad, d), lambda j: (0, 0, 0))],
        out_specs=pl.BlockSpec((2, npad, d), lambda j: (0, 0, 0)),
        out_shape=jax.ShapeDtypeStruct((2, npad, d), jnp.float32),
        scratch_shapes=[pltpu.VMEM((npad, d), jnp.float32)] * 6,
    )(srcb, dstb, tables)


def _tc_scale(degp, h1, h1perm, npad, d, blk=512):
    def body(dinv_ref, h1_ref, hp_ref, t1_ref):
        dv = dinv_ref[:, :]
        t1_ref[0, :, :] = dv * h1_ref[:, :]
        t1_ref[1, :, :] = dv * hp_ref[:, :]
    return pl.pallas_call(
        body,
        grid=(npad // blk,),
        in_specs=[pl.BlockSpec((blk, d), lambda i: (i, 0)),
                  pl.BlockSpec((blk, d), lambda i: (i, 0)),
                  pl.BlockSpec((blk, d), lambda i: (i, 0))],
        out_specs=pl.BlockSpec((2, blk, d), lambda i: (0, i, 0)),
        out_shape=jax.ShapeDtypeStruct((2, npad, d), jnp.float32),
    )(degp, h1, h1perm)


def _tc_l1finish(acc1, dinv, b1, w2, n, npad, d, blk=512):
    def body(acc_ref, dinv_ref, b1_ref, w2_ref, t2_ref):
        i = pl.program_id(0)
        rows = lax.broadcasted_iota(jnp.int32, (blk, d), 0) + i * blk
        m = rows < n
        dv = dinv_ref[:, :]
        for cc in range(2):
            z = jnp.maximum(acc_ref[cc] * dv + b1_ref[:, :], 0.0)
            z = jnp.where(m, z, 0.0)
            t2_ref[cc, :, :] = jnp.dot(z, w2_ref[:, :],
                                       preferred_element_type=jnp.float32) * dv
    return pl.pallas_call(
        body,
        grid=(npad // blk,),
        in_specs=[pl.BlockSpec((2, blk, d), lambda i: (0, i, 0)),
                  pl.BlockSpec((blk, d), lambda i: (i, 0)),
                  pl.BlockSpec((1, d), lambda i: (0, 0)),
                  pl.BlockSpec((d, d), lambda i: (0, 0))],
        out_specs=pl.BlockSpec((2, blk, d), lambda i: (0, i, 0)),
        out_shape=jax.ShapeDtypeStruct((2, npad, d), jnp.float32),
    )(acc1, dinv, b1, w2)


def _tc_proj(acc2, dinv, b2, wd, n, npad, d, blk=512):
    def body(acc_ref, dinv_ref, b2_ref, wd_ref, proj_ref, sum_ref):
        i = pl.program_id(0)

        @pl.when(i == 0)
        def _():
            sum_ref[:, :] = jnp.zeros_like(sum_ref)

        rows = lax.broadcasted_iota(jnp.int32, (blk, d), 0) + i * blk
        z = acc_ref[0] * dinv_ref[:, :] + b2_ref[:, :]
        z = jnp.where(rows < n, z, 0.0)
        sum_ref[:, :] += jnp.sum(z, axis=0, keepdims=True)

        @pl.when(i == pl.num_programs(0) - 1)
        def _():
            summ = jax.nn.sigmoid(sum_ref[:, :] / float(n))
            proj_ref[:, :] = lax.dot_general(
                summ, wd_ref[:, :], (((1,), (1,)), ((), ())),
                preferred_element_type=jnp.float32)
    return pl.pallas_call(
        body,
        grid=(npad // blk,),
        in_specs=[pl.BlockSpec((1, blk, d), lambda i: (0, i, 0)),
                  pl.BlockSpec((blk, d), lambda i: (i, 0)),
                  pl.BlockSpec((1, d), lambda i: (0, 0)),
                  pl.BlockSpec((d, d), lambda i: (0, 0))],
        out_specs=pl.BlockSpec((1, d), lambda i: (0, 0)),
        out_shape=jax.ShapeDtypeStruct((1, d), jnp.float32),
        scratch_shapes=[pltpu.VMEM((1, d), jnp.float32)],
    )(acc2, dinv, b2, wd)


def _tc_loss(acc2, dinv, b2, proj, n, npad, d, blk=512):
    def body(acc_ref, dinv_ref, b2_ref, proj_ref, out_ref, s_ref):
        i = pl.program_id(0)

        @pl.when(i == 0)
        def _():
            s_ref[0] = 0.0
            s_ref[1] = 0.0

        rows = lax.broadcasted_iota(jnp.int32, (blk, 1), 0) + i * blk
        m = rows < n
        dv = dinv_ref[:, :]
        dn = (((1,), (1,)), ((), ()))
        pz = acc_ref[0] * dv + b2_ref[:, :]
        nz = acc_ref[1] * dv + b2_ref[:, :]
        sp = lax.dot_general(pz, proj_ref[:, :], dn,
                             preferred_element_type=jnp.float32)
        sn = lax.dot_general(nz, proj_ref[:, :], dn,
                             preferred_element_type=jnp.float32)
        lp = jnp.where(m, jnp.log(jax.nn.sigmoid(sp) + EPS), 0.0)
        ln = jnp.where(m, jnp.log(1.0 - jax.nn.sigmoid(sn) + EPS), 0.0)
        s_ref[0] += jnp.sum(lp)
        s_ref[1] += jnp.sum(ln)

        @pl.when(i == pl.num_programs(0) - 1)
        def _():
            out_ref[:, :] = jnp.full((1, 1), -(s_ref[0] + s_ref[1]) / float(n),
                                     jnp.float32)
    return pl.pallas_call(
        body,
        grid=(npad // blk,),
        in_specs=[pl.BlockSpec((2, blk, d), lambda i: (0, i, 0)),
                  pl.BlockSpec((blk, d), lambda i: (i, 0)),
                  pl.BlockSpec((1, d), lambda i: (0, 0)),
                  pl.BlockSpec((1, d), lambda i: (0, 0))],
        out_specs=pl.BlockSpec((1, 1), lambda i: (0, 0)),
        out_shape=jax.ShapeDtypeStruct((1, 1), jnp.float32),
        scratch_shapes=[pltpu.SMEM((2,), jnp.float32)],
    )(acc2, dinv, b2, proj)


def kernel(x, edge_index, perm, W1, b1, W2, b2, Wd):
    n, d = x.shape
    e = edge_index.shape[1]
    npad = ((n + 1 + 511) // 512) * 512

    src = edge_index[0].astype(jnp.int32)
    dst = edge_index[1].astype(jnp.int32)

    K = 32768
    ne = ((e + K - 1) // K) * K
    src_p = jnp.pad(src, (0, ne - e), constant_values=n)
    dst_p = jnp.pad(dst, (0, ne - e), constant_values=n)

    perm_p = jnp.pad(perm.astype(jnp.int32), (0, npad - n),
                     constant_values=n)
    x_p = jnp.pad(x, ((0, npad - n), (0, 0)))
    b1r = b1.reshape(1, d)
    b2r = b2.reshape(1, d)

    h1 = _tc_mm(x_p, W1, npad, d)
    dinv = _tc_deg(dst_p, npad, d, ne, K)
    h1perm = _sc_gather(h1, perm_p, npad, d)
    T1 = _tc_scale(dinv, h1, h1perm, npad, d)
    acc1 = _tc_scatter(T1, src_p, dst_p, npad, d, ne, K)
    T2 = _tc_l1finish(acc1, dinv, b1r, W2, n, npad, d)
    acc2 = _tc_scatter(T2, src_p, dst_p, npad, d, ne, K)
    proj = _tc_proj(acc2, dinv, b2r, Wd, n, npad, d)
    out = _tc_loss(acc2, dinv, b2r, proj, n, npad, d)
    return out.reshape(())


# speedup vs baseline: 7.4354x; 1.0502x over previous
import functools

import jax
import jax.numpy as jnp
from jax import lax
from jax.experimental import pallas as pl
from jax.experimental.pallas import tpu as pltpu
from jax.experimental.pallas import tpu_sc as plsc

EPS = 1e-15

NC = 2
NT = 16
CHUNK = 128


def _mesh():
    return plsc.VectorSubcoreMesh(core_axis_name="c", subcore_axis_name="s")


def _sc_gather(table, idx_flat, npad, d):
    rows_per_w = npad // (NC * NT)
    nch = rows_per_w // 64

    @functools.partial(
        pl.kernel, mesh=_mesh(),
        out_type=jax.ShapeDtypeStruct((npad, d), jnp.float32),
        scratch_types=[
            pltpu.VMEM((64,), jnp.int32),
            pltpu.VMEM((64, d), jnp.float32),
            pltpu.SemaphoreType.DMA,
        ],
    )
    def k(tab_hbm, idx_hbm, out_hbm, idx_v, rows_v, sem):
        c = lax.axis_index("c")
        s = lax.axis_index("s")
        wid = c * NT + s

        def body(j, carry):
            pltpu.sync_copy(idx_hbm.at[pl.ds(wid * rows_per_w + j * 64, 64)], idx_v)
            pltpu.async_copy(tab_hbm.at[idx_v], rows_v, sem).wait()
            pltpu.sync_copy(rows_v, out_hbm.at[pl.ds(wid * rows_per_w + j * 64, 64)])
            return carry
        lax.fori_loop(0, nch, body, 0)

    return k(table, idx_flat)


def _tc_mm(x, w, npad, d, blk=1024):
    def body(x_ref, w_ref, o_ref):
        o_ref[:, :] = jnp.dot(x_ref[:, :], w_ref[:, :],
                              preferred_element_type=jnp.float32)
    return pl.pallas_call(
        body,
        grid=(npad // blk,),
        in_specs=[pl.BlockSpec((blk, d), lambda i: (i, 0)),
                  pl.BlockSpec((d, d), lambda i: (0, 0))],
        out_specs=pl.BlockSpec((blk, d), lambda i: (i, 0)),
        out_shape=jax.ShapeDtypeStruct((npad, d), jnp.float32),
    )(x, w)


def _tc_deg(dstb, npad, d, ne, K):
    nch = ne // K

    def body(dst_ref, out_ref, *accs):
        j = pl.program_id(0)

        @pl.when(j == 0)
        def _():
            for a in accs:
                a[:, :] = jnp.zeros_like(a)

        one_row = jnp.full((1, d), 1.0, jnp.float32)
        nw = len(accs)

        def eb(i, carry):
            for w, a in enumerate(accs):
                dw = dst_ref[i + w * (K // nw)]
                a[pl.ds(dw, 1), :] += one_row
            return carry
        lax.fori_loop(0, K // nw, eb, 0, unroll=16)

        @pl.when(j == nch - 1)
        def _():
            tot = accs[0][:, :]
            for a in accs[1:]:
                tot = tot + a[:, :]
            out_ref[:, :] = lax.rsqrt(tot + 1.0)
    return pl.pallas_call(
        body,
        grid=(nch,),
        in_specs=[pl.BlockSpec((K,), lambda j: (j,),
                               memory_space=pltpu.SMEM)],
        out_specs=pl.BlockSpec((npad, d), lambda j: (0, 0)),
        out_shape=jax.ShapeDtypeStruct((npad, d), jnp.float32),
        scratch_shapes=[pltpu.VMEM((npad, d), jnp.float32)] * 8,
    )(dstb)


def _tc_scatter(tables, srcb, dstb, npad, d, ne, K):
    nch = ne // K

    def body(src_ref, dst_ref, tab_ref, out_ref, *accs):
        j = pl.program_id(0)
        nw = len(accs) // 2 + 1

        @pl.when(j == 0)
        def _():
            out_ref[0] = tab_ref[0]
            out_ref[1] = tab_ref[1]
            for a in accs:
                a[:, :] = jnp.zeros_like(a)

        def eb(i, carry):
            for w in range(nw):
                sw = src_ref[i + w * (K // nw)]
                dw = dst_ref[i + w * (K // nw)]
                if w == 0:
                    out_ref[0, pl.ds(dw, 1), :] += tab_ref[0, pl.ds(sw, 1), :]
                    out_ref[1, pl.ds(dw, 1), :] += tab_ref[1, pl.ds(sw, 1), :]
                else:
                    accs[2 * (w - 1)][pl.ds(dw, 1), :] += \
                        tab_ref[0, pl.ds(sw, 1), :]
                    accs[2 * (w - 1) + 1][pl.ds(dw, 1), :] += \
                        tab_ref[1, pl.ds(sw, 1), :]
            return carry
        lax.fori_loop(0, K // nw, eb, 0, unroll=16)

        @pl.when(j == nch - 1)
        def _():
            for c in range(2):
                tot = out_ref[c]
                for w in range(1, nw):
                    tot = tot + accs[2 * (w - 1) + c][:, :]
                out_ref[c] = tot
    return pl.pallas_call(
        body,
        grid=(nch,),
        in_specs=[pl.BlockSpec((K,), lambda j: (j,),
                               memory_space=pltpu.SMEM),
                  pl.BlockSpec((K,), lambda j: (j,),
                               memory_space=pltpu.SMEM),
                  pl.BlockSpec((2, npad, d), lambda j: (0, 0, 0))],
        out_specs=pl.BlockSpec((2, npad, d), lambda j: (0, 0, 0)),
        out_shape=jax.ShapeDtypeStruct((2, npad, d), jnp.float32),
        scratch_shapes=[pltpu.VMEM((npad, d), jnp.float32)] * 6,
    )(srcb, dstb, tables)


def _tc_scale(degp, h1, h1perm, npad, d, blk=512):
    def body(dinv_ref, h1_ref, hp_ref, t1_ref):
        dv = dinv_ref[:, :]
        t1_ref[0, :, :] = dv * h1_ref[:, :]
        t1_ref[1, :, :] = dv * hp_ref[:, :]
    return pl.pallas_call(
        body,
        grid=(npad // blk,),
        in_specs=[pl.BlockSpec((blk, d), lambda i: (i, 0)),
                  pl.BlockSpec((blk, d), lambda i: (i, 0)),
                  pl.BlockSpec((blk, d), lambda i: (i, 0))],
        out_specs=pl.BlockSpec((2, blk, d), lambda i: (0, i, 0)),
        out_shape=jax.ShapeDtypeStruct((2, npad, d), jnp.float32),
    )(degp, h1, h1perm)


def _tc_l1finish(acc1, dinv, b1, w2, n, npad, d, blk=512):
    def body(acc_ref, dinv_ref, b1_ref, w2_ref, t2_ref):
        i = pl.program_id(0)
        rows = lax.broadcasted_iota(jnp.int32, (blk, d), 0) + i * blk
        m = rows < n
        dv = dinv_ref[:, :]
        for cc in range(2):
            z = jnp.maximum(acc_ref[cc] * dv + b1_ref[:, :], 0.0)
            z = jnp.where(m, z, 0.0)
            t2_ref[cc, :, :] = jnp.dot(z, w2_ref[:, :],
                                       preferred_element_type=jnp.float32) * dv
    return pl.pallas_call(
        body,
        grid=(npad // blk,),
        in_specs=[pl.BlockSpec((2, blk, d), lambda i: (0, i, 0)),
                  pl.BlockSpec((blk, d), lambda i: (i, 0)),
                  pl.BlockSpec((1, d), lambda i: (0, 0)),
                  pl.BlockSpec((d, d), lambda i: (0, 0))],
        out_specs=pl.BlockSpec((2, blk, d), lambda i: (0, i, 0)),
        out_shape=jax.ShapeDtypeStruct((2, npad, d), jnp.float32),
    )(acc1, dinv, b1, w2)


def _tc_proj(acc2, dinv, b2, wd, n, npad, d, blk=512):
    def body(acc_ref, dinv_ref, b2_ref, wd_ref, proj_ref, sum_ref):
        i = pl.program_id(0)

        @pl.when(i == 0)
        def _():
            sum_ref[:, :] = jnp.zeros_like(sum_ref)

        rows = lax.broadcasted_iota(jnp.int32, (blk, d), 0) + i * blk
        z = acc_ref[0] * dinv_ref[:, :] + b2_ref[:, :]
        z = jnp.where(rows < n, z, 0.0)
        sum_ref[:, :] += jnp.sum(z, axis=0, keepdims=True)

        @pl.when(i == pl.num_programs(0) - 1)
        def _():
            summ = jax.nn.sigmoid(sum_ref[:, :] / float(n))
            proj_ref[:, :] = lax.dot_general(
                summ, wd_ref[:, :], (((1,), (1,)), ((), ())),
                preferred_element_type=jnp.float32)
    return pl.pallas_call(
        body,
        grid=(npad // blk,),
        in_specs=[pl.BlockSpec((1, blk, d), lambda i: (0, i, 0)),
                  pl.BlockSpec((blk, d), lambda i: (i, 0)),
                  pl.BlockSpec((1, d), lambda i: (0, 0)),
                  pl.BlockSpec((d, d), lambda i: (0, 0))],
        out_specs=pl.BlockSpec((1, d), lambda i: (0, 0)),
        out_shape=jax.ShapeDtypeStruct((1, d), jnp.float32),
        scratch_shapes=[pltpu.VMEM((1, d), jnp.float32)],
    )(acc2, dinv, b2, wd)


def _tc_loss(acc2, dinv, b2, proj, n, npad, d, blk=512):
    def body(acc_ref, dinv_ref, b2_ref, proj_ref, out_ref, s_ref):
        i = pl.program_id(0)

        @pl.when(i == 0)
        def _():
            s_ref[0] = 0.0
            s_ref[1] = 0.0

        rows = lax.broadcasted_iota(jnp.int32, (blk, 1), 0) + i * blk
        m = rows < n
        dv = dinv_ref[:, :]
        dn = (((1,), (1,)), ((), ()))
        pz = acc_ref[0] * dv + b2_ref[:, :]
        nz = acc_ref[1] * dv + b2_ref[:, :]
        sp = lax.dot_general(pz, proj_ref[:, :], dn,
                             preferred_element_type=jnp.float32)
        sn = lax.dot_general(nz, proj_ref[:, :], dn,
                             preferred_element_type=jnp.float32)
        lp = jnp.where(m, jnp.log(jax.nn.sigmoid(sp) + EPS), 0.0)
        ln = jnp.where(m, jnp.log(1.0 - jax.nn.sigmoid(sn) + EPS), 0.0)
        s_ref[0] += jnp.sum(lp)
        s_ref[1] += jnp.sum(ln)

        @pl.when(i == pl.num_programs(0) - 1)
        def _():
            out_ref[:, :] = jnp.full((1, 1), -(s_ref[0] + s_ref[1]) / float(n),
                                     jnp.float32)
    return pl.pallas_call(
        body,
        grid=(npad // blk,),
        in_specs=[pl.BlockSpec((2, blk, d), lambda i: (0, i, 0)),
                  pl.BlockSpec((blk, d), lambda i: (i, 0)),
                  pl.BlockSpec((1, d), lambda i: (0, 0)),
                  pl.BlockSpec((1, d), lambda i: (0, 0))],
        out_specs=pl.BlockSpec((1, 1), lambda i: (0, 0)),
        out_shape=jax.ShapeDtypeStruct((1, 1), jnp.float32),
        scratch_shapes=[pltpu.SMEM((2,), jnp.float32)],
    )(acc2, dinv, b2, proj)


def kernel(x, edge_index, perm, W1, b1, W2, b2, Wd):
    n, d = x.shape
    e = edge_index.shape[1]
    npad = ((n + 1 + 511) // 512) * 512

    src = edge_index[0].astype(jnp.int32)
    dst = edge_index[1].astype(jnp.int32)

    K = 32768
    ne = ((e + K - 1) // K) * K
    src_p = jnp.pad(src, (0, ne - e), constant_values=n)
    dst_p = jnp.pad(dst, (0, ne - e), constant_values=n)

    perm_p = jnp.pad(perm.astype(jnp.int32), (0, npad - n),
                     constant_values=n)
    x_p = jnp.pad(x, ((0, npad - n), (0, 0)))
    b1r = b1.reshape(1, d)
    b2r = b2.reshape(1, d)

    h1 = _tc_mm(x_p, W1, npad, d)
    dinv = _tc_deg(dst_p, npad, d, ne, K)
    h1perm = _sc_gather(h1, perm_p, npad, d)
    T1 = _tc_scale(dinv, h1, h1perm, npad, d)
    acc1 = _tc_scatter(T1, src_p, dst_p, npad, d, ne, K)
    T2 = _tc_l1finish(acc1, dinv, b1r, W2, n, npad, d)
    acc2 = _tc_scatter(T2, src_p, dst_p, npad, d, ne, K)
    proj = _tc_proj(acc2, dinv, b2r, Wd, n, npad, d)
    out = _tc_loss(acc2, dinv, b2r, proj, n, npad, d)
    return out.reshape(())
